# initial kernel scaffold (unmeasured)
import jax
import jax.numpy as jnp
from jax import lax
from jax.experimental import pallas as pl
from jax.experimental.pallas import tpu as pltpu

N_DEV = 32
N_BUF = 2


def kernel(x, w_mat):
    m, k_per = x.shape
    k_full, n = w_mat.shape
    m_blk = m // N_DEV

    def body(x_ref, w_hbm, out_ref, gather_ref, wbuf_ref,
             send_sems, recv_sems, wsems):
        me = lax.axis_index("i")

        barrier_sem = pltpu.get_barrier_semaphore()
        for o in range(1, N_DEV):
            pl.semaphore_signal(
                barrier_sem, inc=1,
                device_id=(lax.rem(me + o, N_DEV),),
                device_id_type=pl.DeviceIdType.MESH,
            )
        pl.semaphore_wait(barrier_sem, N_DEV - 1)

        sends = []
        for o in range(1, N_DEV):
            tgt = lax.rem(me + o, N_DEV)
            rdma = pltpu.make_async_remote_copy(
                src_ref=x_ref.at[pl.ds(tgt * m_blk, m_blk), :],
                dst_ref=gather_ref.at[o],
                send_sem=send_sems.at[o],
                recv_sem=recv_sems.at[o],
                device_id=(tgt,),
                device_id_type=pl.DeviceIdType.MESH,
            )
            rdma.start()
            sends.append(rdma)

        wfetch = {}

        def start_wfetch(o):
            src = lax.rem(me + (N_DEV - o), N_DEV)
            cp = pltpu.make_async_copy(
                w_hbm.at[pl.ds(src * k_per, k_per), :],
                wbuf_ref.at[o % N_BUF],
                wsems.at[o % N_BUF],
            )
            cp.start()
            wfetch[o] = cp

        for o in range(N_BUF):
            start_wfetch(o)

        for o in range(N_DEV):
            slot = o % N_BUF
            wfetch[o].wait()
            if o == 0:
                blk = x_ref[pl.ds(me * m_blk, m_blk), :]
            else:
                pltpu.make_async_remote_copy(
                    src_ref=gather_ref.at[o],
                    dst_ref=gather_ref.at[o],
                    send_sem=send_sems.at[o],
                    recv_sem=recv_sems.at[o],
                    device_id=(me,),
                    device_id_type=pl.DeviceIdType.MESH,
                ).wait_recv()
                blk = gather_ref[o]
            part = jnp.dot(blk, wbuf_ref[slot],
                           preferred_element_type=jnp.float32)
            if o == 0:
                out_ref[...] = part
            elif o == N_DEV - 1:
                out_ref[...] = jnp.maximum(out_ref[...] + part, 0.0)
            else:
                out_ref[...] = out_ref[...] + part
            if o + N_BUF < N_DEV:
                start_wfetch(o + N_BUF)

        for rdma in sends:
            rdma.wait_send()

    return pl.pallas_call(
        body,
        out_shape=jax.ShapeDtypeStruct((m_blk, n), jnp.float32),
        in_specs=[
            pl.BlockSpec(memory_space=pltpu.VMEM),
            pl.BlockSpec(memory_space=pltpu.ANY),
        ],
        out_specs=pl.BlockSpec(memory_space=pltpu.VMEM),
        scratch_shapes=[
            pltpu.VMEM((N_DEV, m_blk, k_per), jnp.float32),
            pltpu.VMEM((N_BUF, k_per, n), jnp.float32),
            pltpu.SemaphoreType.DMA((N_DEV,)),
            pltpu.SemaphoreType.DMA((N_DEV,)),
            pltpu.SemaphoreType.DMA((N_BUF,)),
        ],
        compiler_params=pltpu.CompilerParams(collective_id=0),
    )(x, w_mat)


# baseline (device time: 64654 ns/iter reference)
import jax
import jax.numpy as jnp
from jax import lax
from jax.experimental import pallas as pl
from jax.experimental.pallas import tpu as pltpu

N_DEV = 32
N_BUF = 2


def kernel(x, w_mat):
    m, k_per = x.shape
    k_full, n = w_mat.shape
    m_blk = m // N_DEV

    def body(x_ref, w_hbm, out_ref, gather_ref, wbuf_ref,
             send_sems, recv_sems, wsems):
        me = lax.axis_index("i")

        barrier_sem = pltpu.get_barrier_semaphore()
        for o in range(1, N_DEV):
            pl.semaphore_signal(
                barrier_sem, inc=1,
                device_id=(lax.rem(me + o, N_DEV),),
                device_id_type=pl.DeviceIdType.MESH,
            )
        pl.semaphore_wait(barrier_sem, N_DEV - 1)

        sends = []
        for o in range(1, N_DEV):
            tgt = lax.rem(me + o, N_DEV)
            rdma = pltpu.make_async_remote_copy(
                src_ref=x_ref.at[pl.ds(tgt * m_blk, m_blk), :],
                dst_ref=gather_ref.at[o],
                send_sem=send_sems.at[o],
                recv_sem=recv_sems.at[o],
                device_id=(tgt,),
                device_id_type=pl.DeviceIdType.MESH,
            )
            rdma.start()
            sends.append(rdma)

        wfetch = {}

        def start_wfetch(o):
            src = lax.rem(me + (N_DEV - o), N_DEV)
            cp = pltpu.make_async_copy(
                w_hbm.at[pl.ds(src * k_per, k_per), :],
                wbuf_ref.at[o % N_BUF],
                wsems.at[o % N_BUF],
            )
            cp.start()
            wfetch[o] = cp

        for o in range(N_BUF):
            start_wfetch(o)

        for o in range(N_DEV):
            slot = o % N_BUF
            wfetch[o].wait()
            if o == 0:
                blk = x_ref[pl.ds(me * m_blk, m_blk), :]
            else:
                pltpu.make_async_remote_copy(
                    src_ref=gather_ref.at[o],
                    dst_ref=gather_ref.at[o],
                    send_sem=send_sems.at[o],
                    recv_sem=recv_sems.at[o],
                    device_id=(me,),
                    device_id_type=pl.DeviceIdType.MESH,
                ).wait_recv()
                blk = gather_ref[o]
            part = jnp.dot(blk, wbuf_ref[slot],
                           preferred_element_type=jnp.float32)
            if o == 0:
                out_ref[...] = part
            elif o == N_DEV - 1:
                out_ref[...] = jnp.maximum(out_ref[...] + part, 0.0)
            else:
                out_ref[...] = out_ref[...] + part
            if o + N_BUF < N_DEV:
                start_wfetch(o + N_BUF)

        for rdma in sends:
            rdma.wait_send()

    return pl.pallas_call(
        body,
        out_shape=jax.ShapeDtypeStruct((m_blk, n), jnp.float32),
        in_specs=[
            pl.BlockSpec(memory_space=pltpu.VMEM),
            pl.BlockSpec(memory_space=pl.ANY),
        ],
        out_specs=pl.BlockSpec(memory_space=pltpu.VMEM),
        scratch_shapes=[
            pltpu.VMEM((N_DEV, m_blk, k_per), jnp.float32),
            pltpu.VMEM((N_BUF, k_per, n), jnp.float32),
            pltpu.SemaphoreType.DMA((N_DEV,)),
            pltpu.SemaphoreType.DMA((N_DEV,)),
            pltpu.SemaphoreType.DMA((N_BUF,)),
        ],
        compiler_params=pltpu.CompilerParams(collective_id=0),
    )(x, w_mat)


# device time: 61859 ns/iter; 1.0452x vs baseline; 1.0452x over previous
import jax
import jax.numpy as jnp
from jax import lax
from jax.experimental import pallas as pl
from jax.experimental.pallas import tpu as pltpu

N_DEV = 32
N_BUF = 2
G_OFF = 4
N_GRP = N_DEV // G_OFF


def kernel(x, w_mat):
    m, k_per = x.shape
    k_full, n = w_mat.shape
    m_blk = m // N_DEV
    k_grp = G_OFF * k_per

    def body(x_ref, w_hbm, out_ref, xrow_ref, wbuf_ref,
             send_sems, recv_sems, wsems):
        me = lax.axis_index("i")

        barrier_sem = pltpu.get_barrier_semaphore()
        for o in range(1, N_DEV):
            pl.semaphore_signal(
                barrier_sem, inc=1,
                device_id=(lax.rem(me + o, N_DEV),),
                device_id_type=pl.DeviceIdType.MESH,
            )
        pl.semaphore_wait(barrier_sem, N_DEV - 1)

        sends = []
        for o in range(1, N_DEV):
            tgt = lax.rem(me + o, N_DEV)
            rdma = pltpu.make_async_remote_copy(
                src_ref=x_ref.at[pl.ds(tgt * m_blk, m_blk), :],
                dst_ref=xrow_ref.at[:, pl.ds(o * k_per, k_per)],
                send_sem=send_sems.at[o],
                recv_sem=recv_sems.at[o],
                device_id=(tgt,),
                device_id_type=pl.DeviceIdType.MESH,
            )
            rdma.start()
            sends.append(rdma)

        xrow_ref[:, pl.ds(0, k_per)] = x_ref[pl.ds(me * m_blk, m_blk), :]

        wfetch = {}

        def start_wfetch(g):
            cps = []
            for i in range(G_OFF):
                o = g * G_OFF + i
                src = lax.rem(me + (N_DEV - o), N_DEV)
                cp = pltpu.make_async_copy(
                    w_hbm.at[pl.ds(src * k_per, k_per), :],
                    wbuf_ref.at[g % N_BUF, pl.ds(i * k_per, k_per), :],
                    wsems.at[g % N_BUF],
                )
                cp.start()
                cps.append(cp)
            wfetch[g] = cps

        for g in range(N_BUF):
            start_wfetch(g)

        for g in range(N_GRP):
            slot = g % N_BUF
            for cp in wfetch[g]:
                cp.wait()
            for i in range(G_OFF):
                o = g * G_OFF + i
                if o == 0:
                    continue
                pltpu.make_async_remote_copy(
                    src_ref=xrow_ref.at[:, pl.ds(o * k_per, k_per)],
                    dst_ref=xrow_ref.at[:, pl.ds(o * k_per, k_per)],
                    send_sem=send_sems.at[o],
                    recv_sem=recv_sems.at[o],
                    device_id=(me,),
                    device_id_type=pl.DeviceIdType.MESH,
                ).wait_recv()
            part = jnp.dot(
                xrow_ref[:, pl.ds(g * k_grp, k_grp)],
                wbuf_ref[slot],
                preferred_element_type=jnp.float32,
            )
            if g == 0:
                out_ref[...] = part
            elif g == N_GRP - 1:
                out_ref[...] = jnp.maximum(out_ref[...] + part, 0.0)
            else:
                out_ref[...] = out_ref[...] + part
            if g + N_BUF < N_GRP:
                start_wfetch(g + N_BUF)

        for rdma in sends:
            rdma.wait_send()

    return pl.pallas_call(
        body,
        out_shape=jax.ShapeDtypeStruct((m_blk, n), jnp.float32),
        in_specs=[
            pl.BlockSpec(memory_space=pltpu.VMEM),
            pl.BlockSpec(memory_space=pl.ANY),
        ],
        out_specs=pl.BlockSpec(memory_space=pltpu.VMEM),
        scratch_shapes=[
            pltpu.VMEM((m_blk, k_full), jnp.float32),
            pltpu.VMEM((N_BUF, k_grp, n), jnp.float32),
            pltpu.SemaphoreType.DMA((N_DEV,)),
            pltpu.SemaphoreType.DMA((N_DEV,)),
            pltpu.SemaphoreType.DMA((N_BUF,)),
        ],
        compiler_params=pltpu.CompilerParams(
            collective_id=0,
            vmem_limit_bytes=64 * 1024 * 1024,
        ),
    )(x, w_mat)


# device time: 48994 ns/iter; 1.3196x vs baseline; 1.2626x over previous
import os

import jax
import jax.numpy as jnp
from jax import lax
from jax.experimental import pallas as pl
from jax.experimental.pallas import tpu as pltpu

_ABLATE = os.environ.get("ABLATE", "")

N_DEV = 32
N_BUF = 2
G_OFF = 4
N_GRP = N_DEV // G_OFF


def kernel(x, w_mat):
    m, k_per = x.shape
    k_full, n = w_mat.shape
    m_blk = m // N_DEV
    k_grp = G_OFF * k_per

    def body(x_ref, w_hbm, out_ref, xrow_ref, wbuf_ref,
             send_sems, recv_sems, wsems):
        me = lax.axis_index("i")

        sends = []
        if _ABLATE != "gemm":
            barrier_sem = pltpu.get_barrier_semaphore()
            for o in range(1, N_DEV):
                pl.semaphore_signal(
                    barrier_sem, inc=1,
                    device_id=(lax.rem(me + o, N_DEV),),
                    device_id_type=pl.DeviceIdType.MESH,
                )
            pl.semaphore_wait(barrier_sem, N_DEV - 1)

            for o in range(1, N_DEV):
                tgt = lax.rem(me + o, N_DEV)
                rdma = pltpu.make_async_remote_copy(
                    src_ref=x_ref.at[pl.ds(tgt * m_blk, m_blk), :],
                    dst_ref=xrow_ref.at[:, pl.ds(o * k_per, k_per)],
                    send_sem=send_sems.at[o],
                    recv_sem=recv_sems.at[o],
                    device_id=(tgt,),
                    device_id_type=pl.DeviceIdType.MESH,
                )
                rdma.start()
                sends.append(rdma)

        xrow_ref[:, pl.ds(0, k_per)] = x_ref[pl.ds(me * m_blk, m_blk), :]

        wfetch = {}

        def start_wfetch(g):
            cps = []
            for i in range(G_OFF):
                o = g * G_OFF + i
                src = lax.rem(me + (N_DEV - o), N_DEV)
                cp = pltpu.make_async_copy(
                    w_hbm.at[pl.ds(src * k_per, k_per), :],
                    wbuf_ref.at[g % N_BUF, pl.ds(i * k_per, k_per), :],
                    wsems.at[g % N_BUF],
                )
                cp.start()
                cps.append(cp)
            wfetch[g] = cps

        if _ABLATE != "comm":
            for g in range(N_BUF):
                start_wfetch(g)

        for g in range(N_GRP):
            slot = g % N_BUF
            if _ABLATE != "comm":
                for cp in wfetch[g]:
                    cp.wait()
            if _ABLATE != "gemm":
                for i in range(G_OFF):
                    o = g * G_OFF + i
                    if o == 0:
                        continue
                    pltpu.make_async_remote_copy(
                        src_ref=xrow_ref.at[:, pl.ds(o * k_per, k_per)],
                        dst_ref=xrow_ref.at[:, pl.ds(o * k_per, k_per)],
                        send_sem=send_sems.at[o],
                        recv_sem=recv_sems.at[o],
                        device_id=(me,),
                        device_id_type=pl.DeviceIdType.MESH,
                    ).wait_recv()
            if _ABLATE == "comm":
                continue
            part = jnp.dot(
                xrow_ref[:, pl.ds(g * k_grp, k_grp)],
                wbuf_ref[slot],
                preferred_element_type=jnp.float32,
            )
            if g == 0:
                out_ref[...] = part
            elif g == N_GRP - 1:
                out_ref[...] = jnp.maximum(out_ref[...] + part, 0.0)
            else:
                out_ref[...] = out_ref[...] + part
            if g + N_BUF < N_GRP:
                start_wfetch(g + N_BUF)

        if _ABLATE == "comm":
            out_ref[...] = jnp.zeros_like(out_ref)

        for rdma in sends:
            rdma.wait_send()

    return pl.pallas_call(
        body,
        out_shape=jax.ShapeDtypeStruct((m_blk, n), jnp.float32),
        in_specs=[
            pl.BlockSpec(memory_space=pltpu.VMEM),
            pl.BlockSpec(memory_space=pl.ANY),
        ],
        out_specs=pl.BlockSpec(memory_space=pltpu.VMEM),
        scratch_shapes=[
            pltpu.VMEM((m_blk, k_full), jnp.float32),
            pltpu.VMEM((N_BUF, k_grp, n), jnp.float32),
            pltpu.SemaphoreType.DMA((N_DEV,)),
            pltpu.SemaphoreType.DMA((N_DEV,)),
            pltpu.SemaphoreType.DMA((N_BUF,)),
        ],
        compiler_params=pltpu.CompilerParams(
            collective_id=None if _ABLATE == "gemm" else 0,
            vmem_limit_bytes=64 * 1024 * 1024,
        ),
    )(x, w_mat)
